# baseline (device time: 156094 ns/iter reference)
import jax
import jax.numpy as jnp
from jax import lax
from jax.experimental import pallas as pl
from jax.experimental.pallas import tpu as pltpu

C = 4


def kernel(x):
    xs32 = x[0, 0]
    m, n = xs32.shape
    q = m // 4
    h = q // C

    def body(
        x_hbm,
        out_ref,
        recv_a1,
        recv_b1,
        recv_a2,
        recv_b2,
        send_a,
        send_b,
        mine_a,
        mine_b,
        acc_a,
        acc_b,
        fin_a,
        fin_b,
        pool,
        send_sems,
        recv_sems,
        load_sems,
        store_sems,
    ):
        mx = lax.axis_index("x")
        my = lax.axis_index("y")
        x_nbr = (1 - mx, my)
        y_nbr = (mx, 1 - my)

        a_mine = mx * q
        a_theirs = (1 - mx) * q
        b_mine = 2 * q + my * q
        b_theirs = 2 * q + (1 - my) * q

        barrier = pltpu.get_barrier_semaphore()
        for nbr in (x_nbr, y_nbr):
            pl.semaphore_signal(
                barrier, inc=1, device_id=nbr,
                device_id_type=pl.DeviceIdType.MESH,
            )
        pl.semaphore_wait(barrier, 2)

        def sem_idx(phase, stream, c):
            return phase * 2 * C + stream * C + c

        def exch(src, dst, phase, stream, c, nbr):
            i = sem_idx(phase, stream, c)
            return pltpu.make_async_remote_copy(
                src_ref=src, dst_ref=dst,
                send_sem=send_sems.at[i], recv_sem=recv_sems.at[i],
                device_id=nbr, device_id_type=pl.DeviceIdType.MESH,
            )

        loads = []
        for c in range(C):
            loads.append((a_theirs + c * h, send_a, c))
            loads.append((b_theirs + c * h, send_b, c))
        for c in range(C):
            loads.append((a_mine + c * h, mine_a, c))
            loads.append((b_mine + c * h, mine_b, c))

        def start_load(k):
            row, _, _ = loads[k]
            cp = pltpu.make_async_copy(
                x_hbm.at[pl.ds(row, h), :],
                pool.at[k % 2],
                load_sems.at[k % 2],
            )
            cp.start()
            return cp

        p1a = [
            exch(send_a.at[pl.ds(c * h, h), :],
                 recv_a1.at[pl.ds(c * h, h), :], 0, 0, c, x_nbr)
            for c in range(C)
        ]
        p1b = [
            exch(send_b.at[pl.ds(c * h, h), :],
                 recv_b1.at[pl.ds(c * h, h), :], 0, 1, c, y_nbr)
            for c in range(C)
        ]

        pend = [start_load(0), start_load(1)]
        for k in range(2 * 2 * C):
            pend[k % 2].wait()
            _, dst, c = loads[k]
            dst[pl.ds(c * h, h), :] = pool[k % 2].astype(jnp.bfloat16)
            if k + 2 < 2 * 2 * C:
                pend[k % 2] = start_load(k + 2)
            if k < 2 * C:
                if k % 2 == 0:
                    p1a[k // 2].start()
                else:
                    p1b[k // 2].start()

        p2a = [
            exch(acc_a.at[pl.ds(c * h, h), :],
                 recv_a2.at[pl.ds(c * h, h), :], 1, 0, c, y_nbr)
            for c in range(C)
        ]
        p2b = [
            exch(acc_b.at[pl.ds(c * h, h), :],
                 recv_b2.at[pl.ds(c * h, h), :], 1, 1, c, x_nbr)
            for c in range(C)
        ]
        for c in range(C):
            s = pl.ds(c * h, h)
            p1a[c].wait()
            acc_a[s, :] = mine_a[s, :] + recv_a1[s, :]
            p2a[c].start()
            p1b[c].wait()
            acc_b[s, :] = mine_b[s, :] + recv_b1[s, :]
            p2b[c].start()

        p3 = []
        stores = []
        for c in range(C):
            s = pl.ds(c * h, h)
            ra = pl.ds(a_mine + c * h, h)
            rb = pl.ds(b_mine + c * h, h)
            p2a[c].wait()
            fin_a[s, :] = acc_a[s, :] + recv_a2[s, :]
            r = exch(fin_a.at[s, :], out_ref.at[ra, :], 2, 0, c, x_nbr)
            r.start()
            p3.append(r)
            cp = pltpu.make_async_copy(
                fin_a.at[s, :], out_ref.at[ra, :], store_sems.at[2 * c]
            )
            cp.start()
            stores.append(cp)
            p2b[c].wait()
            fin_b[s, :] = acc_b[s, :] + recv_b2[s, :]
            r = exch(fin_b.at[s, :], out_ref.at[rb, :], 2, 1, c, y_nbr)
            r.start()
            p3.append(r)
            cp = pltpu.make_async_copy(
                fin_b.at[s, :], out_ref.at[rb, :], store_sems.at[2 * c + 1]
            )
            cp.start()
            stores.append(cp)
        for cp in stores:
            cp.wait()
        for r in p3:
            r.wait()

    return pl.pallas_call(
        body,
        out_shape=jax.ShapeDtypeStruct((m, n), jnp.bfloat16),
        in_specs=[pl.BlockSpec(memory_space=pl.ANY)],
        out_specs=pl.BlockSpec(memory_space=pl.ANY),
        scratch_shapes=[
            pltpu.VMEM((q, n), jnp.bfloat16),
            pltpu.VMEM((q, n), jnp.bfloat16),
            pltpu.VMEM((q, n), jnp.bfloat16),
            pltpu.VMEM((q, n), jnp.bfloat16),
            pltpu.VMEM((q, n), jnp.bfloat16),
            pltpu.VMEM((q, n), jnp.bfloat16),
            pltpu.VMEM((q, n), jnp.bfloat16),
            pltpu.VMEM((q, n), jnp.bfloat16),
            pltpu.VMEM((q, n), jnp.bfloat16),
            pltpu.VMEM((q, n), jnp.bfloat16),
            pltpu.VMEM((q, n), jnp.bfloat16),
            pltpu.VMEM((q, n), jnp.bfloat16),
            pltpu.VMEM((2, h, n), jnp.float32),
            pltpu.SemaphoreType.DMA((3 * 2 * C,)),
            pltpu.SemaphoreType.DMA((3 * 2 * C,)),
            pltpu.SemaphoreType.DMA((2,)),
            pltpu.SemaphoreType.DMA((2 * C,)),
        ],
        compiler_params=pltpu.CompilerParams(
            collective_id=0,
            vmem_limit_bytes=100 * 1024 * 1024,
        ),
    )(xs32)
